# baseline (speedup 1.0000x reference)
import jax
import jax.numpy as jnp
from jax.experimental import pallas as pl
from jax.experimental.pallas import tpu as pltpu

MAX_WIDTH = 0.1
N_MASKS = 2
MASK_VALUE = 0.0

_BT = 128


def _mask_kernel(src_ref, full_ref, starts_ref, ends_ref, x_ref, o_ref):
    b = pl.program_id(0)
    tb = pl.program_id(1)
    full = full_ref[b, tb]

    @pl.when(full == 1)
    def _():
        o_ref[...] = jnp.full(o_ref.shape, MASK_VALUE, o_ref.dtype)

    @pl.when(full == 0)
    def _():
        rows = tb * _BT + jax.lax.broadcasted_iota(jnp.int32, (1, _BT, 1), 1)
        masked = jnp.zeros(rows.shape, dtype=jnp.bool_)
        for m in range(N_MASKS):
            s = starts_ref[b, m]
            e = ends_ref[b, m]
            masked = masked | ((rows >= s) & (rows < e))
        o_ref[...] = jnp.where(masked, jnp.float32(MASK_VALUE), x_ref[...])


def _spans(B, T):
    kw, ks = jax.random.split(jax.random.key(1))
    max_w = int(MAX_WIDTH * T)
    widths = jax.random.randint(kw, (B, N_MASKS), 1, max_w + 1)
    starts = jax.random.randint(ks, (B, N_MASKS), 0, T)
    starts = jnp.minimum(starts, T - widths)
    return starts.astype(jnp.int32), (starts + widths).astype(jnp.int32)


@jax.jit
def kernel(x):
    B, T, F = x.shape
    starts, ends = _spans(B, T)
    nb = T // _BT
    t = jnp.arange(T, dtype=jnp.int32)
    rowmask = jnp.zeros((B, T), dtype=jnp.bool_)
    for m in range(N_MASKS):
        rowmask = rowmask | (
            (t[None, :] >= starts[:, m : m + 1]) & (t[None, :] < ends[:, m : m + 1])
        )
    full = jnp.all(rowmask.reshape(B, nb, _BT), axis=-1)
    idx = jnp.arange(nb, dtype=jnp.int32)[None, :]
    src = jax.lax.cummax(jnp.where(full, jnp.int32(-1), idx), axis=1)
    src = jnp.maximum(src, 0)
    return pl.pallas_call(
        _mask_kernel,
        grid_spec=pltpu.PrefetchScalarGridSpec(
            num_scalar_prefetch=4,
            grid=(B, nb),
            in_specs=[
                pl.BlockSpec(
                    (1, _BT, F), lambda b, j, src, fl, s0, s1: (b, src[b, j], 0)
                ),
            ],
            out_specs=pl.BlockSpec((1, _BT, F), lambda b, j, src, fl, s0, s1: (b, j, 0)),
        ),
        out_shape=jax.ShapeDtypeStruct(x.shape, x.dtype),
    )(src, full.astype(jnp.int32), starts, ends, x)
